# baseline (speedup 1.0000x reference)
import functools

import jax
import jax.numpy as jnp
from jax import lax
from jax.experimental import pallas as pl
from jax.experimental.pallas import tpu as pltpu
from jax.experimental.pallas import tpu_sc as plsc

_N = 8388608
_NC = 2
_NS = 16
_NW = _NC * _NS
_L = 16
_C = 16384
_NBUF = 2
_PER_W = _N // _NW
_NCHUNK = _PER_W // _C


def _sc_body(x_hbm, y_hbm, out_hbm, part_hbm, *refs):
    xb = refs[0:_NBUF]
    yb = refs[_NBUF:2 * _NBUF]
    ob = refs[2 * _NBUF:3 * _NBUF]
    pb = refs[3 * _NBUF]
    sl = refs[3 * _NBUF + 1:3 * _NBUF + 1 + _NBUF]
    ss = refs[3 * _NBUF + 1 + _NBUF:]

    wid = lax.axis_index("s") * _NC + lax.axis_index("c")
    base = wid * _PER_W

    zero = jnp.zeros((_L,), jnp.float32)

    def start_load(g, b):
        off = base + g * _C
        pltpu.make_async_copy(x_hbm.at[pl.ds(off, _C)], xb[b], sl[b]).start()
        pltpu.make_async_copy(y_hbm.at[pl.ds(off, _C)], yb[b], sl[b]).start()

    def wait_load(b):
        pltpu.make_async_copy(x_hbm.at[pl.ds(0, _C)], xb[b], sl[b]).wait()
        pltpu.make_async_copy(y_hbm.at[pl.ds(0, _C)], yb[b], sl[b]).wait()

    def start_store(g, b):
        off = base + g * _C
        pltpu.make_async_copy(ob[b], out_hbm.at[pl.ds(off, _C)], ss[b]).start()

    def wait_store(b):
        pltpu.make_async_copy(ob[b], out_hbm.at[pl.ds(0, _C)], ss[b]).wait()

    def compute(b, carry):
        xr, yr, orr = xb[b], yb[b], ob[b]

        @plsc.parallel_loop(0, _C // _L, carry=carry, unroll=8)
        def body(i, cc):
            s, q = cc
            xv = xr[pl.ds(i * _L, _L)]
            yv = yr[pl.ds(i * _L, _L)]
            ov = xv + yv
            orr[pl.ds(i * _L, _L)] = ov
            iv = ov.astype(jnp.int32)
            s = s + (iv & 1)
            q = q + (iv >> 1)
            return (s, q)

        return body

    for b in range(_NBUF):
        start_load(b, b)

    def jbody(j, carry):
        for b in range(_NBUF):
            g = _NBUF * j + b
            wait_load(b)

            @pl.when(j >= 1)
            def _():
                wait_store(b)

            carry = compute(b, carry)
            start_store(g, b)

            @pl.when(j < _NCHUNK // _NBUF - 1)
            def _():
                start_load(g + _NBUF, b)
        return carry

    zero_i = jnp.zeros((_L,), jnp.int32)
    s, q = lax.fori_loop(0, _NCHUNK // _NBUF, jbody, (zero_i, zero_i))
    for b in range(_NBUF):
        wait_store(b)

    pb[pl.ds(0, _L)] = s.astype(jnp.float32)
    pb[pl.ds(_L, _L)] = q.astype(jnp.float32)
    pltpu.sync_copy(pb, part_hbm.at[wid])


_sc_call = functools.partial(
    pl.kernel,
    out_type=[
        jax.ShapeDtypeStruct((_N,), jnp.float32),
        jax.ShapeDtypeStruct((_NW, 2 * _L), jnp.float32),
    ],
    mesh=plsc.VectorSubcoreMesh(core_axis_name="c", subcore_axis_name="s"),
    scratch_types=(
        [pltpu.VMEM((_C,), jnp.float32)] * (3 * _NBUF)
        + [pltpu.VMEM((2 * _L,), jnp.float32)]
        + [pltpu.SemaphoreType.DMA] * (2 * _NBUF)
    ),
)(_sc_body)


@jax.jit
def kernel(x, y):
    out, parts = _sc_call(x, y)
    parts_i = parts.astype(jnp.int32)
    n1 = jnp.sum(parts_i[:, :_L])
    n2 = jnp.sum(parts_i[:, _L:])
    n0 = _N - n1 - n2
    c0 = n0.astype(jnp.float32)
    c2 = n2.astype(jnp.float32)
    n = jnp.float32(_N)
    denom0 = n - c2
    denom1 = n - c0
    valid0 = denom0 > 0
    valid1 = denom1 > 0
    iou0 = jnp.where(valid0, c0 / jnp.where(valid0, denom0, 1.0), 0.0)
    iou1 = jnp.where(valid1, c2 / jnp.where(valid1, denom1, 1.0), 0.0)
    n_valid = jnp.maximum(
        valid0.astype(jnp.float32) + valid1.astype(jnp.float32), 1.0
    )
    miou = (iou0 + iou1) / n_valid
    return out, miou


# speedup vs baseline: 1.3181x; 1.3181x over previous
import functools

import jax
import jax.numpy as jnp
from jax import lax
from jax.experimental import pallas as pl
from jax.experimental.pallas import tpu as pltpu
from jax.experimental.pallas import tpu_sc as plsc

_N = 8388608
_NC = 2
_NS = 16
_NW = _NC * _NS
_L = 16
_C = 16384
_NBUF = 2
_PER_W = _N // _NW
_NCHUNK = _PER_W // _C


def _sc_body(x_hbm, y_hbm, out_hbm, part_hbm, *refs):
    xb = refs[0:_NBUF]
    yb = refs[_NBUF:2 * _NBUF]
    ob = refs[2 * _NBUF:3 * _NBUF]
    pb = refs[3 * _NBUF]
    acc_s = refs[3 * _NBUF + 1]
    acc_q = refs[3 * _NBUF + 2]
    sl = refs[3 * _NBUF + 3:3 * _NBUF + 3 + _NBUF]
    ss = refs[3 * _NBUF + 3 + _NBUF:]

    wid = lax.axis_index("s") * _NC + lax.axis_index("c")
    base = wid * _PER_W

    zero = jnp.zeros((_L,), jnp.float32)

    def start_load(g, b):
        off = base + g * _C
        pltpu.make_async_copy(x_hbm.at[pl.ds(off, _C)], xb[b], sl[b]).start()
        pltpu.make_async_copy(y_hbm.at[pl.ds(off, _C)], yb[b], sl[b]).start()

    def wait_load(b):
        pltpu.make_async_copy(x_hbm.at[pl.ds(0, _C)], xb[b], sl[b]).wait()
        pltpu.make_async_copy(y_hbm.at[pl.ds(0, _C)], yb[b], sl[b]).wait()

    def start_store(g, b):
        off = base + g * _C
        pltpu.make_async_copy(ob[b], out_hbm.at[pl.ds(off, _C)], ss[b]).start()

    def wait_store(b):
        pltpu.make_async_copy(ob[b], out_hbm.at[pl.ds(0, _C)], ss[b]).wait()

    def compute(b):
        xr, yr, orr = xb[b], yb[b], ob[b]

        @plsc.parallel_loop(0, _C // _L, unroll=8)
        def body(i):
            xv = xr[pl.ds(i * _L, _L)]
            yv = yr[pl.ds(i * _L, _L)]
            ov = xv + yv
            orr[pl.ds(i * _L, _L)] = ov
            plsc.addupdate(acc_s.at[...], (1.0 - xv) * (1.0 - yv))
            plsc.addupdate(acc_q.at[...], xv * yv)

    acc_s[pl.ds(0, _L)] = zero
    acc_q[pl.ds(0, _L)] = zero

    for b in range(_NBUF):
        start_load(b, b)

    def jbody(j, _):
        for b in range(_NBUF):
            g = _NBUF * j + b
            wait_load(b)

            @pl.when(j >= 1)
            def _():
                wait_store(b)

            compute(b)
            start_store(g, b)

            @pl.when(j < _NCHUNK // _NBUF - 1)
            def _():
                start_load(g + _NBUF, b)
        return 0

    lax.fori_loop(0, _NCHUNK // _NBUF, jbody, 0)
    for b in range(_NBUF):
        wait_store(b)

    pb[pl.ds(0, _L)] = acc_s[pl.ds(0, _L)]
    pb[pl.ds(_L, _L)] = acc_q[pl.ds(0, _L)]
    pltpu.sync_copy(pb, part_hbm.at[wid])


_sc_call = functools.partial(
    pl.kernel,
    out_type=[
        jax.ShapeDtypeStruct((_N,), jnp.float32),
        jax.ShapeDtypeStruct((_NW, 2 * _L), jnp.float32),
    ],
    mesh=plsc.VectorSubcoreMesh(core_axis_name="c", subcore_axis_name="s"),
    scratch_types=(
        [pltpu.VMEM((_C,), jnp.float32)] * (3 * _NBUF)
        + [pltpu.VMEM((2 * _L,), jnp.float32)]
        + [pltpu.VMEM((_L,), jnp.float32)] * 2
        + [pltpu.SemaphoreType.DMA] * (2 * _NBUF)
    ),
)(_sc_body)


@jax.jit
def kernel(x, y):
    out, parts = _sc_call(x, y)
    c0 = jnp.sum(parts[:, :_L])
    c2 = jnp.sum(parts[:, _L:])
    n = jnp.float32(_N)
    denom0 = n - c2
    denom1 = n - c0
    valid0 = denom0 > 0
    valid1 = denom1 > 0
    iou0 = jnp.where(valid0, c0 / jnp.where(valid0, denom0, 1.0), 0.0)
    iou1 = jnp.where(valid1, c2 / jnp.where(valid1, denom1, 1.0), 0.0)
    n_valid = jnp.maximum(
        valid0.astype(jnp.float32) + valid1.astype(jnp.float32), 1.0
    )
    miou = (iou0 + iou1) / n_valid
    return out, miou


# speedup vs baseline: 1.3300x; 1.0090x over previous
import functools

import jax
import jax.numpy as jnp
from jax import lax
from jax.experimental import pallas as pl
from jax.experimental.pallas import tpu as pltpu
from jax.experimental.pallas import tpu_sc as plsc

_N = 8388608
_NC = 2
_NS = 16
_NW = _NC * _NS
_L = 16
_C = 16384
_NBUF = 2
_PER_W = _N // _NW
_NCHUNK = _PER_W // _C


def _sc_body(x_hbm, y_hbm, out_hbm, part_hbm, *refs):
    xb = refs[0:_NBUF]
    yb = refs[_NBUF:2 * _NBUF]
    ob = refs[2 * _NBUF:3 * _NBUF]
    pb = refs[3 * _NBUF]
    acc_s = refs[3 * _NBUF + 1]
    acc_q = refs[3 * _NBUF + 2]
    sl = refs[3 * _NBUF + 3:3 * _NBUF + 3 + _NBUF]
    ss = refs[3 * _NBUF + 3 + _NBUF:]

    wid = lax.axis_index("s") * _NC + lax.axis_index("c")
    base = wid * _PER_W

    zero = jnp.zeros((_L,), jnp.float32)

    def start_load(g, b):
        off = base + g * _C
        pltpu.make_async_copy(x_hbm.at[pl.ds(off, _C)], xb[b], sl[b]).start()
        pltpu.make_async_copy(y_hbm.at[pl.ds(off, _C)], yb[b], sl[b]).start()

    def wait_load(b):
        pltpu.make_async_copy(x_hbm.at[pl.ds(0, _C)], xb[b], sl[b]).wait()
        pltpu.make_async_copy(y_hbm.at[pl.ds(0, _C)], yb[b], sl[b]).wait()

    def start_store(g, b):
        off = base + g * _C
        pltpu.make_async_copy(ob[b], out_hbm.at[pl.ds(off, _C)], ss[b]).start()

    def wait_store(b):
        pltpu.make_async_copy(ob[b], out_hbm.at[pl.ds(0, _C)], ss[b]).wait()

    def compute(b):
        xr, yr, orr = xb[b], yb[b], ob[b]

        @plsc.parallel_loop(0, _C // _L, unroll=8)
        def body(i):
            xv = xr[pl.ds(i * _L, _L)]
            yv = yr[pl.ds(i * _L, _L)]
            ov = xv + yv
            orr[pl.ds(i * _L, _L)] = ov
            plsc.addupdate(acc_s.at[...], ov)
            plsc.addupdate(acc_q.at[...], xv * yv)

    acc_s[pl.ds(0, _L)] = zero
    acc_q[pl.ds(0, _L)] = zero

    for b in range(_NBUF):
        start_load(b, b)

    def jbody(j, _):
        for b in range(_NBUF):
            g = _NBUF * j + b
            wait_load(b)

            @pl.when(j >= 1)
            def _():
                wait_store(b)

            compute(b)
            start_store(g, b)

            @pl.when(j < _NCHUNK // _NBUF - 1)
            def _():
                start_load(g + _NBUF, b)
        return 0

    lax.fori_loop(0, _NCHUNK // _NBUF, jbody, 0)
    for b in range(_NBUF):
        wait_store(b)

    pb[pl.ds(0, _L)] = acc_s[pl.ds(0, _L)]
    pb[pl.ds(_L, _L)] = acc_q[pl.ds(0, _L)]
    pltpu.sync_copy(pb, part_hbm.at[wid])


_sc_call = functools.partial(
    pl.kernel,
    out_type=[
        jax.ShapeDtypeStruct((_N,), jnp.float32),
        jax.ShapeDtypeStruct((_NW, 2 * _L), jnp.float32),
    ],
    mesh=plsc.VectorSubcoreMesh(core_axis_name="c", subcore_axis_name="s"),
    scratch_types=(
        [pltpu.VMEM((_C,), jnp.float32)] * (3 * _NBUF)
        + [pltpu.VMEM((2 * _L,), jnp.float32)]
        + [pltpu.VMEM((_L,), jnp.float32)] * 2
        + [pltpu.SemaphoreType.DMA] * (2 * _NBUF)
    ),
)(_sc_body)


@jax.jit
def kernel(x, y):
    out, parts = _sc_call(x, y)
    s_tot = jnp.sum(parts[:, :_L])
    c2 = jnp.sum(parts[:, _L:])
    n = jnp.float32(_N)
    c0 = n - s_tot + c2
    denom0 = n - c2
    denom1 = n - c0
    valid0 = denom0 > 0
    valid1 = denom1 > 0
    iou0 = jnp.where(valid0, c0 / jnp.where(valid0, denom0, 1.0), 0.0)
    iou1 = jnp.where(valid1, c2 / jnp.where(valid1, denom1, 1.0), 0.0)
    n_valid = jnp.maximum(
        valid0.astype(jnp.float32) + valid1.astype(jnp.float32), 1.0
    )
    miou = (iou0 + iou1) / n_valid
    return out, miou


# speedup vs baseline: 1.3578x; 1.0209x over previous
import functools

import jax
import jax.numpy as jnp
from jax import lax
from jax.experimental import pallas as pl
from jax.experimental.pallas import tpu as pltpu
from jax.experimental.pallas import tpu_sc as plsc

_N = 8388608
_NC = 2
_NS = 16
_NW = _NC * _NS
_L = 16
_C = 16384
_NBUF = 2
_PER_W = _N // _NW
_NCHUNK = _PER_W // _C


_UB = 8


def _sc_body(x_hbm, y_hbm, out_hbm, part_hbm, *refs):
    xb = refs[0:_NBUF]
    yb = refs[_NBUF:2 * _NBUF]
    ob = refs[2 * _NBUF:3 * _NBUF]
    acc_s = refs[3 * _NBUF]
    acc_q = refs[3 * _NBUF + 1]
    sl = refs[3 * _NBUF + 2:3 * _NBUF + 2 + _NBUF]
    ss = refs[3 * _NBUF + 2 + _NBUF:]

    wid = lax.axis_index("s") * _NC + lax.axis_index("c")
    base = wid * _PER_W

    zero = jnp.zeros((_L,), jnp.float32)

    def start_load(g, b):
        off = base + g * _C
        pltpu.make_async_copy(x_hbm.at[pl.ds(off, _C)], xb[b], sl[b]).start()
        pltpu.make_async_copy(y_hbm.at[pl.ds(off, _C)], yb[b], sl[b]).start()

    def wait_load(b):
        pltpu.make_async_copy(x_hbm.at[pl.ds(0, _C)], xb[b], sl[b]).wait()
        pltpu.make_async_copy(y_hbm.at[pl.ds(0, _C)], yb[b], sl[b]).wait()

    def start_store(g, b):
        off = base + g * _C
        pltpu.make_async_copy(ob[b], out_hbm.at[pl.ds(off, _C)], ss[b]).start()

    def wait_store(b):
        pltpu.make_async_copy(ob[b], out_hbm.at[pl.ds(0, _C)], ss[b]).wait()

    def compute(b):
        xr, yr, orr = xb[b], yb[b], ob[b]

        @plsc.parallel_loop(0, _C // (_L * _UB))
        def body(i):
            for k in range(_UB):
                off = i * (_L * _UB) + k * _L
                xv = xr[pl.ds(off, _L)]
                yv = yr[pl.ds(off, _L)]
                ov = xv + yv
                orr[pl.ds(off, _L)] = ov
                plsc.addupdate(acc_s.at[pl.ds(k * _L, _L)], ov)
                plsc.addupdate(acc_q.at[pl.ds(k * _L, _L)], xv * yv)

    for k in range(_UB):
        acc_s[pl.ds(k * _L, _L)] = zero
        acc_q[pl.ds(k * _L, _L)] = zero

    for b in range(_NBUF):
        start_load(b, b)

    def jbody(j, _):
        for b in range(_NBUF):
            g = _NBUF * j + b
            wait_load(b)

            @pl.when(j >= 1)
            def _():
                wait_store(b)

            compute(b)
            start_store(g, b)

            @pl.when(j < _NCHUNK // _NBUF - 1)
            def _():
                start_load(g + _NBUF, b)
        return 0

    lax.fori_loop(0, _NCHUNK // _NBUF, jbody, 0)
    for b in range(_NBUF):
        wait_store(b)

    pltpu.sync_copy(acc_s, part_hbm.at[wid, pl.ds(0, _UB * _L)])
    pltpu.sync_copy(acc_q, part_hbm.at[wid, pl.ds(_UB * _L, _UB * _L)])


_sc_call = functools.partial(
    pl.kernel,
    out_type=[
        jax.ShapeDtypeStruct((_N,), jnp.float32),
        jax.ShapeDtypeStruct((_NW, 2 * _UB * _L), jnp.float32),
    ],
    mesh=plsc.VectorSubcoreMesh(core_axis_name="c", subcore_axis_name="s"),
    scratch_types=(
        [pltpu.VMEM((_C,), jnp.float32)] * (3 * _NBUF)
        + [pltpu.VMEM((_UB * _L,), jnp.float32)] * 2
        + [pltpu.SemaphoreType.DMA] * (2 * _NBUF)
    ),
)(_sc_body)


@jax.jit
def kernel(x, y):
    out, parts = _sc_call(x, y)
    s_tot = jnp.sum(parts[:, :_UB * _L])
    c2 = jnp.sum(parts[:, _UB * _L:])
    n = jnp.float32(_N)
    c0 = n - s_tot + c2
    denom0 = n - c2
    denom1 = n - c0
    valid0 = denom0 > 0
    valid1 = denom1 > 0
    iou0 = jnp.where(valid0, c0 / jnp.where(valid0, denom0, 1.0), 0.0)
    iou1 = jnp.where(valid1, c2 / jnp.where(valid1, denom1, 1.0), 0.0)
    n_valid = jnp.maximum(
        valid0.astype(jnp.float32) + valid1.astype(jnp.float32), 1.0
    )
    miou = (iou0 + iou1) / n_valid
    return out, miou


# speedup vs baseline: 1.4997x; 1.1045x over previous
import functools

import jax
import jax.numpy as jnp
from jax import lax
from jax.experimental import pallas as pl
from jax.experimental.pallas import tpu as pltpu
from jax.experimental.pallas import tpu_sc as plsc

_N = 8388608
_NC = 2
_NS = 16
_NW = _NC * _NS
_L = 16
_C = 16384
_NBUF = 2
_PER_W = _N // _NW
_NCHUNK = _PER_W // _C


_UB = 8


def _sc_body(x_hbm, y_hbm, out_hbm, part_hbm, *refs):
    xb = refs[0:_NBUF]
    yb = refs[_NBUF:2 * _NBUF]
    ob = refs[2 * _NBUF:3 * _NBUF]
    acc_s = refs[3 * _NBUF]
    acc_q = refs[3 * _NBUF + 1]
    sl = refs[3 * _NBUF + 2:3 * _NBUF + 2 + _NBUF]
    ss = refs[3 * _NBUF + 2 + _NBUF:]

    wid = lax.axis_index("s") * _NC + lax.axis_index("c")
    base = wid * _PER_W

    zero = jnp.zeros((_L,), jnp.float32)

    def start_load(g, b):
        off = base + g * _C
        pltpu.make_async_copy(x_hbm.at[pl.ds(off, _C)], xb[b], sl[b]).start()
        pltpu.make_async_copy(y_hbm.at[pl.ds(off, _C)], yb[b], sl[b]).start()

    def wait_load(b):
        pltpu.make_async_copy(x_hbm.at[pl.ds(0, _C)], xb[b], sl[b]).wait()
        pltpu.make_async_copy(y_hbm.at[pl.ds(0, _C)], yb[b], sl[b]).wait()

    def start_store(g, b):
        off = base + g * _C
        pltpu.make_async_copy(ob[b], out_hbm.at[pl.ds(off, _C)], ss[b]).start()

    def wait_store(b):
        pltpu.make_async_copy(ob[b], out_hbm.at[pl.ds(0, _C)], ss[b]).wait()

    def compute(b):
        xr, yr, orr = xb[b], yb[b], ob[b]

        @plsc.parallel_loop(0, _C // (_L * _UB))
        def body(i):
            s_loc = zero
            q_loc = zero
            for k in range(_UB):
                off = i * (_L * _UB) + k * _L
                xv = xr[pl.ds(off, _L)]
                yv = yr[pl.ds(off, _L)]
                ov = xv + yv
                orr[pl.ds(off, _L)] = ov
                s_loc = s_loc + ov
                q_loc = q_loc + xv * yv
            plsc.addupdate(acc_s.at[pl.ds(0, _L)], s_loc)
            plsc.addupdate(acc_q.at[pl.ds(0, _L)], q_loc)

    for k in range(_UB):
        acc_s[pl.ds(k * _L, _L)] = zero
        acc_q[pl.ds(k * _L, _L)] = zero

    for b in range(_NBUF):
        start_load(b, b)

    def jbody(j, _):
        for b in range(_NBUF):
            g = _NBUF * j + b
            wait_load(b)

            @pl.when(j >= 1)
            def _():
                wait_store(b)

            compute(b)
            start_store(g, b)

            @pl.when(j < _NCHUNK // _NBUF - 1)
            def _():
                start_load(g + _NBUF, b)
        return 0

    lax.fori_loop(0, _NCHUNK // _NBUF, jbody, 0)
    for b in range(_NBUF):
        wait_store(b)

    pltpu.sync_copy(acc_s, part_hbm.at[wid, pl.ds(0, _UB * _L)])
    pltpu.sync_copy(acc_q, part_hbm.at[wid, pl.ds(_UB * _L, _UB * _L)])


_sc_call = functools.partial(
    pl.kernel,
    out_type=[
        jax.ShapeDtypeStruct((_N,), jnp.float32),
        jax.ShapeDtypeStruct((_NW, 2 * _UB * _L), jnp.float32),
    ],
    mesh=plsc.VectorSubcoreMesh(core_axis_name="c", subcore_axis_name="s"),
    scratch_types=(
        [pltpu.VMEM((_C,), jnp.float32)] * (3 * _NBUF)
        + [pltpu.VMEM((_UB * _L,), jnp.float32)] * 2
        + [pltpu.SemaphoreType.DMA] * (2 * _NBUF)
    ),
)(_sc_body)


@jax.jit
def kernel(x, y):
    out, parts = _sc_call(x, y)
    s_tot = jnp.sum(parts[:, :_UB * _L])
    c2 = jnp.sum(parts[:, _UB * _L:])
    n = jnp.float32(_N)
    c0 = n - s_tot + c2
    denom0 = n - c2
    denom1 = n - c0
    valid0 = denom0 > 0
    valid1 = denom1 > 0
    iou0 = jnp.where(valid0, c0 / jnp.where(valid0, denom0, 1.0), 0.0)
    iou1 = jnp.where(valid1, c2 / jnp.where(valid1, denom1, 1.0), 0.0)
    n_valid = jnp.maximum(
        valid0.astype(jnp.float32) + valid1.astype(jnp.float32), 1.0
    )
    miou = (iou0 + iou1) / n_valid
    return out, miou


# speedup vs baseline: 1.5053x; 1.0037x over previous
import functools

import jax
import jax.numpy as jnp
from jax import lax
from jax.experimental import pallas as pl
from jax.experimental.pallas import tpu as pltpu
from jax.experimental.pallas import tpu_sc as plsc

_N = 8388608
_NC = 2
_NS = 16
_NW = _NC * _NS
_L = 16
_C = 16384
_NBUF = 2
_PER_W = _N // _NW
_NCHUNK = _PER_W // _C


_UB = 8


def _sc_body(x_hbm, y_hbm, out_hbm, part_hbm, *refs):
    xb = refs[0:_NBUF]
    yb = refs[_NBUF:2 * _NBUF]
    ob = refs[2 * _NBUF:3 * _NBUF]
    acc_s = refs[3 * _NBUF]
    acc_q = refs[3 * _NBUF + 1]
    sl = refs[3 * _NBUF + 2:3 * _NBUF + 2 + _NBUF]
    ss = refs[3 * _NBUF + 2 + _NBUF:]

    wid = lax.axis_index("s") * _NC + lax.axis_index("c")
    base = wid * _PER_W

    zero = jnp.zeros((_L,), jnp.float32)

    def start_load(g, b):
        off = base + g * _C
        pltpu.make_async_copy(x_hbm.at[pl.ds(off, _C)], xb[b], sl[b]).start()
        pltpu.make_async_copy(y_hbm.at[pl.ds(off, _C)], yb[b], sl[b]).start()

    def wait_load(b):
        pltpu.make_async_copy(x_hbm.at[pl.ds(0, _C)], xb[b], sl[b]).wait()
        pltpu.make_async_copy(y_hbm.at[pl.ds(0, _C)], yb[b], sl[b]).wait()

    def start_store(g, b):
        off = base + g * _C
        pltpu.make_async_copy(ob[b], out_hbm.at[pl.ds(off, _C)], ss[b]).start()

    def wait_store(b):
        pltpu.make_async_copy(ob[b], out_hbm.at[pl.ds(0, _C)], ss[b]).wait()

    def compute(b):
        xr, yr, orr = xb[b], yb[b], ob[b]

        @plsc.parallel_loop(0, _C // (_L * _UB))
        def body(i):
            s0 = zero
            s1 = zero
            q0 = zero
            q1 = zero
            for k in range(0, _UB, 2):
                off = i * (_L * _UB) + k * _L
                xv = xr[pl.ds(off, _L)]
                yv = yr[pl.ds(off, _L)]
                xw = xr[pl.ds(off + _L, _L)]
                yw = yr[pl.ds(off + _L, _L)]
                ov = xv + yv
                ow = xw + yw
                orr[pl.ds(off, _L)] = ov
                orr[pl.ds(off + _L, _L)] = ow
                s0 = s0 + ov
                s1 = s1 + ow
                q0 = q0 + xv * yv
                q1 = q1 + xw * yw
            plsc.addupdate(acc_s.at[pl.ds(0, _L)], s0 + s1)
            plsc.addupdate(acc_q.at[pl.ds(0, _L)], q0 + q1)

    for k in range(_UB):
        acc_s[pl.ds(k * _L, _L)] = zero
        acc_q[pl.ds(k * _L, _L)] = zero

    for b in range(_NBUF):
        start_load(b, b)

    def jbody(j, _):
        for b in range(_NBUF):
            g = _NBUF * j + b
            wait_load(b)

            @pl.when(j >= 1)
            def _():
                wait_store(b)

            compute(b)
            start_store(g, b)

            @pl.when(j < _NCHUNK // _NBUF - 1)
            def _():
                start_load(g + _NBUF, b)
        return 0

    lax.fori_loop(0, _NCHUNK // _NBUF, jbody, 0)
    for b in range(_NBUF):
        wait_store(b)

    pltpu.sync_copy(acc_s, part_hbm.at[wid, pl.ds(0, _UB * _L)])
    pltpu.sync_copy(acc_q, part_hbm.at[wid, pl.ds(_UB * _L, _UB * _L)])


_sc_call = functools.partial(
    pl.kernel,
    out_type=[
        jax.ShapeDtypeStruct((_N,), jnp.float32),
        jax.ShapeDtypeStruct((_NW, 2 * _UB * _L), jnp.float32),
    ],
    mesh=plsc.VectorSubcoreMesh(core_axis_name="c", subcore_axis_name="s"),
    scratch_types=(
        [pltpu.VMEM((_C,), jnp.float32)] * (3 * _NBUF)
        + [pltpu.VMEM((_UB * _L,), jnp.float32)] * 2
        + [pltpu.SemaphoreType.DMA] * (2 * _NBUF)
    ),
)(_sc_body)


@jax.jit
def kernel(x, y):
    out, parts = _sc_call(x, y)
    s_tot = jnp.sum(parts[:, :_UB * _L])
    c2 = jnp.sum(parts[:, _UB * _L:])
    n = jnp.float32(_N)
    c0 = n - s_tot + c2
    denom0 = n - c2
    denom1 = n - c0
    valid0 = denom0 > 0
    valid1 = denom1 > 0
    iou0 = jnp.where(valid0, c0 / jnp.where(valid0, denom0, 1.0), 0.0)
    iou1 = jnp.where(valid1, c2 / jnp.where(valid1, denom1, 1.0), 0.0)
    n_valid = jnp.maximum(
        valid0.astype(jnp.float32) + valid1.astype(jnp.float32), 1.0
    )
    miou = (iou0 + iou1) / n_valid
    return out, miou
